# baseline (speedup 1.0000x reference)
import functools

import jax
import jax.numpy as jnp
from jax import lax
from jax.experimental import pallas as pl
from jax.experimental.pallas import tpu as pltpu
from jax.experimental.pallas import tpu_sc as plsc

M0 = 0.3
SIZE = (40, 40)
LIMS = [[1.0, -1.0], [-1.0, 1.0]]
N = SIZE[0] * SIZE[1]
B = 8

SC_SPLIT = 448
TC_ROWS = 288

NW = 32
LANES = 16
J = N // LANES


def _grid_points():
    e0 = jnp.linspace(LIMS[0][0], LIMS[0][1], SIZE[0])
    e1 = jnp.linspace(LIMS[1][0], LIMS[1][1], SIZE[1])
    g = jnp.stack([jnp.tile(e1, SIZE[0]), jnp.repeat(e0, SIZE[1])], axis=1)
    return g.astype(jnp.float32)


def _tc_body(rows, y0_ref, y1_ref, xt_ref, w_ref, o_ref):
    y0 = y0_ref[:, :]
    y1 = y1_ref[:, :]
    x0 = xt_ref[0, 0:1, :]
    x1 = xt_ref[0, 1:2, :]
    w = w_ref[0, :, :]

    d0 = y0 - x0
    d1 = y1 - x1
    dist = jnp.sqrt(d0 * d0 + d1 * d1)
    t = dist * dist
    t_bits = lax.bitcast_convert_type(t, jnp.int32)

    wb = M0 * jnp.sum(w)

    lo0 = jnp.min(t_bits, axis=1, keepdims=True) - 1
    hi0 = jnp.max(t_bits, axis=1, keepdims=True)

    def body(_, carry):
        lo, hi = carry
        mid = lo + ((hi - lo) >> 1)
        cnt = jnp.sum(jnp.where(t_bits <= mid, w, 0.0), axis=1,
                      keepdims=True)
        pred = cnt >= wb
        return jnp.where(pred, lo, mid), jnp.where(pred, mid, hi)

    lo, hi = lax.fori_loop(0, 20, body, (lo0, hi0))

    t_star = lax.bitcast_convert_type(hi, jnp.float32)
    mask = t_bits < hi
    s_w = jnp.sum(jnp.where(mask, w, 0.0), axis=1, keepdims=True)
    s_d = jnp.sum(jnp.where(mask, w * t, 0.0), axis=1, keepdims=True)
    val = jnp.maximum(s_d + t_star * (wb - s_w), 0.0)
    o_ref[0, :, :] = jnp.sqrt(val / wb)


def _tc_call(y0, y1, xt, weight, n_tc):
    nb = n_tc // TC_ROWS
    out = pl.pallas_call(
        functools.partial(_tc_body, TC_ROWS),
        grid=(B, nb),
        in_specs=[
            pl.BlockSpec((TC_ROWS, 1), lambda b, rb: (rb, 0)),
            pl.BlockSpec((TC_ROWS, 1), lambda b, rb: (rb, 0)),
            pl.BlockSpec((1, 2, N), lambda b, rb: (b, 0, 0)),
            pl.BlockSpec((1, 1, N), lambda b, rb: (b, 0, 0)),
        ],
        out_specs=pl.BlockSpec((1, TC_ROWS, 1), lambda b, rb: (b, rb, 0)),
        out_shape=jax.ShapeDtypeStruct((B, n_tc, 1), jnp.float32),
    )(y0[:n_tc], y1[:n_tc], xt, weight.reshape(B, 1, N))
    return out.reshape(B, n_tc)


def _lane_sum(x):
    iota = lax.broadcasted_iota(jnp.int32, (LANES,), 0)
    dnums = lax.GatherDimensionNumbers(
        offset_dims=(), collapsed_slice_dims=(0,), start_index_map=(0,))
    for k in (8, 4, 2, 1):
        perm = lax.gather(
            x, (iota ^ k)[:, None], dimension_numbers=dnums,
            slice_sizes=(1,), mode=lax.GatherScatterMode.PROMISE_IN_BOUNDS)
        x = x + perm
    return x


def _splat(vec, lane):
    dnums = lax.GatherDimensionNumbers(
        offset_dims=(), collapsed_slice_dims=(0,), start_index_map=(0,))
    idx = jnp.full((LANES, 1), lane, jnp.int32)
    return lax.gather(vec, idx, dimension_numbers=dnums, slice_sizes=(1,),
                      mode=lax.GatherScatterMode.PROMISE_IN_BOUNDS)


SC_U = 64
SC_ITERS = 18


def _sc_call(x0, x1, w, y0f, y1f, split):
    rpw = split // 4
    tc_rows = N - split
    groups = rpw // LANES
    nch = N // SC_U
    mesh = plsc.VectorSubcoreMesh(core_axis_name="c", subcore_axis_name="s")

    @functools.partial(
        pl.kernel, mesh=mesh,
        out_type=jax.ShapeDtypeStruct((NW, rpw), jnp.float32),
        scratch_types=[
            pltpu.VMEM((N,), jnp.float32),
            pltpu.VMEM((N,), jnp.float32),
            pltpu.VMEM((N,), jnp.float32),
            pltpu.VMEM((N * LANES,), jnp.int32),
            pltpu.VMEM((rpw,), jnp.float32),
            pltpu.VMEM((rpw,), jnp.float32),
            pltpu.VMEM((rpw,), jnp.float32),
        ],
    )
    def sc_kernel(x0h, x1h, wh, y0h, y1h, oh, x0v, x1v, wv, tbv, y0v, y1v, ov):
        wid = lax.axis_index("s") * 2 + lax.axis_index("c")
        b = wid // 4
        roff = tc_rows + (wid % 4) * rpw
        pltpu.sync_copy(x0h.at[b], x0v)
        pltpu.sync_copy(x1h.at[b], x1v)
        pltpu.sync_copy(wh.at[b], wv)
        pltpu.sync_copy(y0h.at[pl.ds(roff, rpw)], y0v)
        pltpu.sync_copy(y1h.at[pl.ds(roff, rpw)], y1v)

        acc = jnp.zeros((LANES,), jnp.float32)
        for j in range(J):
            acc = acc + wv[pl.ds(j * LANES, LANES)]
        wb = M0 * _lane_sum(acc)

        def group_body(g, _):
            gbase = pl.multiple_of(g * LANES, LANES)
            y0r = y0v[pl.ds(gbase, LANES)]
            y1r = y1v[pl.ds(gbase, LANES)]

            def dist_body(c, mm):
                tmin, tmax = mm
                cb16 = pl.multiple_of(c * SC_U, SC_U)
                cbT = pl.multiple_of(c * (SC_U * LANES), SC_U * LANES)
                x0c = [x0v[pl.ds(cb16 + h * LANES, LANES)]
                       for h in range(SC_U // LANES)]
                x1c = [x1v[pl.ds(cb16 + h * LANES, LANES)]
                       for h in range(SC_U // LANES)]
                for u in range(SC_U):
                    uu = u % LANES
                    x0s = _splat(x0c[u // LANES], uu)
                    x1s = _splat(x1c[u // LANES], uu)
                    d0 = y0r - x0s
                    d1 = y1r - x1s
                    t = d0 * d0 + d1 * d1
                    tmin = jnp.minimum(tmin, t)
                    tmax = jnp.maximum(tmax, t)
                    tbv[pl.ds(cbT + u * LANES, LANES)] = (
                        lax.bitcast_convert_type(t, jnp.int32))
                return tmin, tmax

            big = jnp.full((LANES,), 3.4e38, jnp.float32)
            zz = jnp.zeros((LANES,), jnp.float32)
            tmin, tmax = lax.fori_loop(0, nch, dist_body, (big, zz))

            lo0 = lax.bitcast_convert_type(tmin, jnp.int32) - 1
            hi0 = lax.bitcast_convert_type(tmax, jnp.int32)

            def bis(_, lh):
                lo, hi = lh
                mid = lo + ((hi - lo) >> 1)

                def cnt_body(c, a):
                    cb16 = pl.multiple_of(c * SC_U, SC_U)
                    cbT = pl.multiple_of(c * (SC_U * LANES), SC_U * LANES)
                    wc = [wv[pl.ds(cb16 + h * LANES, LANES)]
                          for h in range(SC_U // LANES)]
                    for u in range(SC_U):
                        ws = _splat(wc[u // LANES], u % LANES)
                        tb = tbv[pl.ds(cbT + u * LANES, LANES)]
                        a = a + jnp.where(tb <= mid, ws, 0.0)
                    return a

                cnt = lax.fori_loop(
                    0, nch, cnt_body, jnp.zeros((LANES,), jnp.float32))
                pred = cnt >= wb
                return (jnp.where(pred, lo, mid), jnp.where(pred, mid, hi))

            lo, hi = lax.fori_loop(0, SC_ITERS, bis, (lo0, hi0))

            def tail_body(c, ab):
                aw, ad = ab
                cb16 = pl.multiple_of(c * SC_U, SC_U)
                cbT = pl.multiple_of(c * (SC_U * LANES), SC_U * LANES)
                wc = [wv[pl.ds(cb16 + h * LANES, LANES)]
                      for h in range(SC_U // LANES)]
                for u in range(SC_U):
                    ws = _splat(wc[u // LANES], u % LANES)
                    tb = tbv[pl.ds(cbT + u * LANES, LANES)]
                    msk = tb < hi
                    tf = lax.bitcast_convert_type(tb, jnp.float32)
                    aw = aw + jnp.where(msk, ws, 0.0)
                    ad = ad + jnp.where(msk, ws * tf, 0.0)
                return aw, ad

            aw, ad = lax.fori_loop(0, nch, tail_body, (zz, zz))
            t_star = lax.bitcast_convert_type(hi, jnp.float32)
            val = jnp.maximum(ad + t_star * (wb - aw), 0.0)
            v = val / wb
            vb = lax.bitcast_convert_type(v, jnp.int32)
            s = lax.bitcast_convert_type((vb >> 1) + 0x1FBD1DF5, jnp.float32)
            for _ in range(4):
                s = 0.5 * (s + v / s)
            ov[pl.ds(gbase, LANES)] = jnp.where(v > 0, s, 0.0)
            return 0

        lax.fori_loop(0, groups, group_body, 0)
        pltpu.sync_copy(ov, oh.at[wid])

    out = sc_kernel(x0, x1, w, y0f, y1f)
    return out.reshape(B, split)


def kernel(input, weight):
    g = _grid_points()
    y0 = g[:, 0:1]
    y1 = g[:, 1:2]
    parts = []
    if SC_SPLIT < N:
        xt = jnp.swapaxes(input, 1, 2)
        parts.append(_tc_call(y0, y1, xt, weight, N - SC_SPLIT))
    if SC_SPLIT > 0:
        x0 = input[:, :, 0]
        x1 = input[:, :, 1]
        parts.append(_sc_call(x0, x1, weight, g[:, 0], g[:, 1], SC_SPLIT))
    if len(parts) == 1:
        return parts[0]
    return jnp.concatenate(parts, axis=1)


# speedup vs baseline: 1.0178x; 1.0178x over previous
import functools

import jax
import jax.numpy as jnp
from jax import lax
from jax.experimental import pallas as pl
from jax.experimental.pallas import tpu as pltpu
from jax.experimental.pallas import tpu_sc as plsc

M0 = 0.3
SIZE = (40, 40)
LIMS = [[1.0, -1.0], [-1.0, 1.0]]
N = SIZE[0] * SIZE[1]
B = 8

SC_SPLIT = 384
TC_ROWS = 304

NW = 32
LANES = 16
J = N // LANES


def _grid_points():
    e0 = jnp.linspace(LIMS[0][0], LIMS[0][1], SIZE[0])
    e1 = jnp.linspace(LIMS[1][0], LIMS[1][1], SIZE[1])
    g = jnp.stack([jnp.tile(e1, SIZE[0]), jnp.repeat(e0, SIZE[1])], axis=1)
    return g.astype(jnp.float32)


def _tc_body(rows, y0_ref, y1_ref, xt_ref, w_ref, o_ref):
    y0 = y0_ref[:, :]
    y1 = y1_ref[:, :]
    x0 = xt_ref[0, 0:1, :]
    x1 = xt_ref[0, 1:2, :]
    w = w_ref[0, :, :]

    d0 = y0 - x0
    d1 = y1 - x1
    dist = jnp.sqrt(d0 * d0 + d1 * d1)
    t = dist * dist
    t_bits = lax.bitcast_convert_type(t, jnp.int32)

    wb = M0 * jnp.sum(w)

    lo0 = jnp.min(t_bits, axis=1, keepdims=True) - 1
    hi0 = jnp.max(t_bits, axis=1, keepdims=True)

    def body(_, carry):
        lo, hi = carry
        mid = lo + ((hi - lo) >> 1)
        cnt = jnp.sum(jnp.where(t_bits <= mid, w, 0.0), axis=1,
                      keepdims=True)
        pred = cnt >= wb
        return jnp.where(pred, lo, mid), jnp.where(pred, mid, hi)

    lo, hi = lax.fori_loop(0, 20, body, (lo0, hi0))

    t_star = lax.bitcast_convert_type(hi, jnp.float32)
    mask = t_bits < hi
    s_w = jnp.sum(jnp.where(mask, w, 0.0), axis=1, keepdims=True)
    s_d = jnp.sum(jnp.where(mask, w * t, 0.0), axis=1, keepdims=True)
    val = jnp.maximum(s_d + t_star * (wb - s_w), 0.0)
    o_ref[0, :, :] = jnp.sqrt(val / wb)


def _tc_call(y0, y1, xt, weight, n_tc):
    nb = n_tc // TC_ROWS
    out = pl.pallas_call(
        functools.partial(_tc_body, TC_ROWS),
        grid=(B, nb),
        in_specs=[
            pl.BlockSpec((TC_ROWS, 1), lambda b, rb: (rb, 0)),
            pl.BlockSpec((TC_ROWS, 1), lambda b, rb: (rb, 0)),
            pl.BlockSpec((1, 2, N), lambda b, rb: (b, 0, 0)),
            pl.BlockSpec((1, 1, N), lambda b, rb: (b, 0, 0)),
        ],
        out_specs=pl.BlockSpec((1, TC_ROWS, 1), lambda b, rb: (b, rb, 0)),
        out_shape=jax.ShapeDtypeStruct((B, n_tc, 1), jnp.float32),
    )(y0[:n_tc], y1[:n_tc], xt, weight.reshape(B, 1, N))
    return out.reshape(B, n_tc)


def _lane_sum(x):
    iota = lax.broadcasted_iota(jnp.int32, (LANES,), 0)
    dnums = lax.GatherDimensionNumbers(
        offset_dims=(), collapsed_slice_dims=(0,), start_index_map=(0,))
    for k in (8, 4, 2, 1):
        perm = lax.gather(
            x, (iota ^ k)[:, None], dimension_numbers=dnums,
            slice_sizes=(1,), mode=lax.GatherScatterMode.PROMISE_IN_BOUNDS)
        x = x + perm
    return x


def _splat(vec, lane):
    dnums = lax.GatherDimensionNumbers(
        offset_dims=(), collapsed_slice_dims=(0,), start_index_map=(0,))
    idx = jnp.full((LANES, 1), lane, jnp.int32)
    return lax.gather(vec, idx, dimension_numbers=dnums, slice_sizes=(1,),
                      mode=lax.GatherScatterMode.PROMISE_IN_BOUNDS)


SC_U = 32
SC_ITERS = 18


def _sc_call(x0, x1, w, y0f, y1f, split):
    rpw = split // 4
    tc_rows = N - split
    groups = rpw // LANES
    nch = N // SC_U
    mesh = plsc.VectorSubcoreMesh(core_axis_name="c", subcore_axis_name="s")

    @functools.partial(
        pl.kernel, mesh=mesh,
        out_type=jax.ShapeDtypeStruct((NW, rpw), jnp.float32),
        scratch_types=[
            pltpu.VMEM((N,), jnp.float32),
            pltpu.VMEM((N,), jnp.float32),
            pltpu.VMEM((N,), jnp.float32),
            pltpu.VMEM((N * LANES,), jnp.int32),
            pltpu.VMEM((rpw,), jnp.float32),
            pltpu.VMEM((rpw,), jnp.float32),
            pltpu.VMEM((rpw,), jnp.float32),
        ],
    )
    def sc_kernel(x0h, x1h, wh, y0h, y1h, oh, x0v, x1v, wv, tbv, y0v, y1v, ov):
        wid = lax.axis_index("s") * 2 + lax.axis_index("c")
        b = wid // 4
        roff = tc_rows + (wid % 4) * rpw
        pltpu.sync_copy(x0h.at[b], x0v)
        pltpu.sync_copy(x1h.at[b], x1v)
        pltpu.sync_copy(wh.at[b], wv)
        pltpu.sync_copy(y0h.at[pl.ds(roff, rpw)], y0v)
        pltpu.sync_copy(y1h.at[pl.ds(roff, rpw)], y1v)

        acc = jnp.zeros((LANES,), jnp.float32)
        for j in range(J):
            acc = acc + wv[pl.ds(j * LANES, LANES)]
        wb = M0 * _lane_sum(acc)

        def group_body(g, _):
            gbase = pl.multiple_of(g * LANES, LANES)
            y0r = y0v[pl.ds(gbase, LANES)]
            y1r = y1v[pl.ds(gbase, LANES)]

            def dist_body(c, mm):
                tmin, tmax = mm
                cb16 = pl.multiple_of(c * SC_U, SC_U)
                cbT = pl.multiple_of(c * (SC_U * LANES), SC_U * LANES)
                x0c = [x0v[pl.ds(cb16 + h * LANES, LANES)]
                       for h in range(SC_U // LANES)]
                x1c = [x1v[pl.ds(cb16 + h * LANES, LANES)]
                       for h in range(SC_U // LANES)]
                for u in range(SC_U):
                    uu = u % LANES
                    x0s = _splat(x0c[u // LANES], uu)
                    x1s = _splat(x1c[u // LANES], uu)
                    d0 = y0r - x0s
                    d1 = y1r - x1s
                    t = d0 * d0 + d1 * d1
                    tmin = jnp.minimum(tmin, t)
                    tmax = jnp.maximum(tmax, t)
                    tbv[pl.ds(cbT + u * LANES, LANES)] = (
                        lax.bitcast_convert_type(t, jnp.int32))
                return tmin, tmax

            big = jnp.full((LANES,), 3.4e38, jnp.float32)
            zz = jnp.zeros((LANES,), jnp.float32)
            tmin, tmax = lax.fori_loop(0, nch, dist_body, (big, zz))

            lo0 = lax.bitcast_convert_type(tmin, jnp.int32) - 1
            hi0 = lax.bitcast_convert_type(tmax, jnp.int32)

            def bis(_, lh):
                lo, hi = lh
                mid = lo + ((hi - lo) >> 1)

                def cnt_body(c, a):
                    cb16 = pl.multiple_of(c * SC_U, SC_U)
                    cbT = pl.multiple_of(c * (SC_U * LANES), SC_U * LANES)
                    wc = [wv[pl.ds(cb16 + h * LANES, LANES)]
                          for h in range(SC_U // LANES)]
                    for u in range(SC_U):
                        ws = _splat(wc[u // LANES], u % LANES)
                        tb = tbv[pl.ds(cbT + u * LANES, LANES)]
                        a = a + jnp.where(tb <= mid, ws, 0.0)
                    return a

                cnt = lax.fori_loop(
                    0, nch, cnt_body, jnp.zeros((LANES,), jnp.float32))
                pred = cnt >= wb
                return (jnp.where(pred, lo, mid), jnp.where(pred, mid, hi))

            lo, hi = lax.fori_loop(0, SC_ITERS, bis, (lo0, hi0))

            def tail_body(c, ab):
                aw, ad = ab
                cb16 = pl.multiple_of(c * SC_U, SC_U)
                cbT = pl.multiple_of(c * (SC_U * LANES), SC_U * LANES)
                wc = [wv[pl.ds(cb16 + h * LANES, LANES)]
                      for h in range(SC_U // LANES)]
                for u in range(SC_U):
                    ws = _splat(wc[u // LANES], u % LANES)
                    tb = tbv[pl.ds(cbT + u * LANES, LANES)]
                    msk = tb < hi
                    tf = lax.bitcast_convert_type(tb, jnp.float32)
                    aw = aw + jnp.where(msk, ws, 0.0)
                    ad = ad + jnp.where(msk, ws * tf, 0.0)
                return aw, ad

            aw, ad = lax.fori_loop(0, nch, tail_body, (zz, zz))
            t_star = lax.bitcast_convert_type(hi, jnp.float32)
            val = jnp.maximum(ad + t_star * (wb - aw), 0.0)
            v = val / wb
            vb = lax.bitcast_convert_type(v, jnp.int32)
            s = lax.bitcast_convert_type((vb >> 1) + 0x1FBD1DF5, jnp.float32)
            for _ in range(4):
                s = 0.5 * (s + v / s)
            ov[pl.ds(gbase, LANES)] = jnp.where(v > 0, s, 0.0)
            return 0

        lax.fori_loop(0, groups, group_body, 0)
        pltpu.sync_copy(ov, oh.at[wid])

    out = sc_kernel(x0, x1, w, y0f, y1f)
    return out.reshape(B, split)


def kernel(input, weight):
    g = _grid_points()
    y0 = g[:, 0:1]
    y1 = g[:, 1:2]
    parts = []
    if SC_SPLIT < N:
        xt = jnp.swapaxes(input, 1, 2)
        parts.append(_tc_call(y0, y1, xt, weight, N - SC_SPLIT))
    if SC_SPLIT > 0:
        x0 = input[:, :, 0]
        x1 = input[:, :, 1]
        parts.append(_sc_call(x0, x1, weight, g[:, 0], g[:, 1], SC_SPLIT))
    if len(parts) == 1:
        return parts[0]
    return jnp.concatenate(parts, axis=1)


# speedup vs baseline: 1.0856x; 1.0666x over previous
import functools

import jax
import jax.numpy as jnp
from jax import lax
from jax.experimental import pallas as pl
from jax.experimental.pallas import tpu as pltpu
from jax.experimental.pallas import tpu_sc as plsc

M0 = 0.3
SIZE = (40, 40)
LIMS = [[1.0, -1.0], [-1.0, 1.0]]
N = SIZE[0] * SIZE[1]
B = 8

SC_SPLIT = 448
TC_ROWS = 384

NW = 32
LANES = 16
J = N // LANES


def _grid_points():
    e0 = jnp.linspace(LIMS[0][0], LIMS[0][1], SIZE[0])
    e1 = jnp.linspace(LIMS[1][0], LIMS[1][1], SIZE[1])
    g = jnp.stack([jnp.tile(e1, SIZE[0]), jnp.repeat(e0, SIZE[1])], axis=1)
    return g.astype(jnp.float32)


def _tc_body(rows, y0_ref, y1_ref, xt_ref, w_ref, o_ref):
    y0 = y0_ref[:, :]
    y1 = y1_ref[:, :]
    x0 = xt_ref[0, 0:1, :]
    x1 = xt_ref[0, 1:2, :]
    w = w_ref[0, :, :]

    d0 = y0 - x0
    d1 = y1 - x1
    dist = jnp.sqrt(d0 * d0 + d1 * d1)
    t = dist * dist
    t_bits = lax.bitcast_convert_type(t, jnp.int32)

    wb = M0 * jnp.sum(w)

    lo0 = jnp.min(t_bits, axis=1, keepdims=True) - 1
    hi0 = jnp.max(t_bits, axis=1, keepdims=True)

    def body(_, carry):
        lo, hi = carry
        mid = lo + ((hi - lo) >> 1)
        cnt = jnp.sum(jnp.where(t_bits <= mid, w, 0.0), axis=1,
                      keepdims=True)
        pred = cnt >= wb
        return jnp.where(pred, lo, mid), jnp.where(pred, mid, hi)

    lo, hi = lax.fori_loop(0, 20, body, (lo0, hi0))

    t_star = lax.bitcast_convert_type(hi, jnp.float32)
    mask = t_bits < hi
    s_w = jnp.sum(jnp.where(mask, w, 0.0), axis=1, keepdims=True)
    s_d = jnp.sum(jnp.where(mask, w * t, 0.0), axis=1, keepdims=True)
    val = jnp.maximum(s_d + t_star * (wb - s_w), 0.0)
    o_ref[0, :, :] = jnp.sqrt(val / wb)


def _tc_call(y0, y1, xt, weight, n_tc):
    nb = n_tc // TC_ROWS
    out = pl.pallas_call(
        functools.partial(_tc_body, TC_ROWS),
        grid=(B, nb),
        in_specs=[
            pl.BlockSpec((TC_ROWS, 1), lambda b, rb: (rb, 0)),
            pl.BlockSpec((TC_ROWS, 1), lambda b, rb: (rb, 0)),
            pl.BlockSpec((1, 2, N), lambda b, rb: (b, 0, 0)),
            pl.BlockSpec((1, 1, N), lambda b, rb: (b, 0, 0)),
        ],
        out_specs=pl.BlockSpec((1, TC_ROWS, 1), lambda b, rb: (b, rb, 0)),
        out_shape=jax.ShapeDtypeStruct((B, n_tc, 1), jnp.float32),
    )(y0[:n_tc], y1[:n_tc], xt, weight.reshape(B, 1, N))
    return out.reshape(B, n_tc)


def _lane_sum(x):
    iota = lax.broadcasted_iota(jnp.int32, (LANES,), 0)
    dnums = lax.GatherDimensionNumbers(
        offset_dims=(), collapsed_slice_dims=(0,), start_index_map=(0,))
    for k in (8, 4, 2, 1):
        perm = lax.gather(
            x, (iota ^ k)[:, None], dimension_numbers=dnums,
            slice_sizes=(1,), mode=lax.GatherScatterMode.PROMISE_IN_BOUNDS)
        x = x + perm
    return x


def _splat(vec, lane):
    dnums = lax.GatherDimensionNumbers(
        offset_dims=(), collapsed_slice_dims=(0,), start_index_map=(0,))
    idx = jnp.full((LANES, 1), lane, jnp.int32)
    return lax.gather(vec, idx, dimension_numbers=dnums, slice_sizes=(1,),
                      mode=lax.GatherScatterMode.PROMISE_IN_BOUNDS)


SC_U = 32
SC_ITERS = 18


def _sc_call(x0, x1, w, y0f, y1f, split):
    rpw = split // 4
    tc_rows = N - split
    groups = rpw // LANES
    nch = N // SC_U
    mesh = plsc.VectorSubcoreMesh(core_axis_name="c", subcore_axis_name="s")

    @functools.partial(
        pl.kernel, mesh=mesh,
        out_type=jax.ShapeDtypeStruct((NW, rpw), jnp.float32),
        scratch_types=[
            pltpu.VMEM((N,), jnp.float32),
            pltpu.VMEM((N,), jnp.float32),
            pltpu.VMEM((N,), jnp.float32),
            pltpu.VMEM((N * LANES,), jnp.int32),
            pltpu.VMEM((rpw,), jnp.float32),
            pltpu.VMEM((rpw,), jnp.float32),
            pltpu.VMEM((rpw,), jnp.float32),
        ],
    )
    def sc_kernel(x0h, x1h, wh, y0h, y1h, oh, x0v, x1v, wv, tbv, y0v, y1v, ov):
        wid = lax.axis_index("s") * 2 + lax.axis_index("c")
        b = wid // 4
        roff = tc_rows + (wid % 4) * rpw
        pltpu.sync_copy(x0h.at[b], x0v)
        pltpu.sync_copy(x1h.at[b], x1v)
        pltpu.sync_copy(wh.at[b], wv)
        pltpu.sync_copy(y0h.at[pl.ds(roff, rpw)], y0v)
        pltpu.sync_copy(y1h.at[pl.ds(roff, rpw)], y1v)

        acc = jnp.zeros((LANES,), jnp.float32)
        for j in range(J):
            acc = acc + wv[pl.ds(j * LANES, LANES)]
        wb = M0 * _lane_sum(acc)

        def group_body(g, _):
            gbase = pl.multiple_of(g * LANES, LANES)
            y0r = y0v[pl.ds(gbase, LANES)]
            y1r = y1v[pl.ds(gbase, LANES)]

            def dist_body(c, mm):
                tmin, tmax = mm
                cb16 = pl.multiple_of(c * SC_U, SC_U)
                cbT = pl.multiple_of(c * (SC_U * LANES), SC_U * LANES)
                x0c = [x0v[pl.ds(cb16 + h * LANES, LANES)]
                       for h in range(SC_U // LANES)]
                x1c = [x1v[pl.ds(cb16 + h * LANES, LANES)]
                       for h in range(SC_U // LANES)]
                for u in range(SC_U):
                    uu = u % LANES
                    x0s = _splat(x0c[u // LANES], uu)
                    x1s = _splat(x1c[u // LANES], uu)
                    d0 = y0r - x0s
                    d1 = y1r - x1s
                    t = d0 * d0 + d1 * d1
                    tmin = jnp.minimum(tmin, t)
                    tmax = jnp.maximum(tmax, t)
                    tbv[pl.ds(cbT + u * LANES, LANES)] = (
                        lax.bitcast_convert_type(t, jnp.int32))
                return tmin, tmax

            big = jnp.full((LANES,), 3.4e38, jnp.float32)
            zz = jnp.zeros((LANES,), jnp.float32)
            tmin, tmax = lax.fori_loop(0, nch, dist_body, (big, zz))

            lo0 = lax.bitcast_convert_type(tmin, jnp.int32) - 1
            hi0 = lax.bitcast_convert_type(tmax, jnp.int32)

            def bis(_, lh):
                lo, hi = lh
                mid = lo + ((hi - lo) >> 1)

                def cnt_body(c, a):
                    cb16 = pl.multiple_of(c * SC_U, SC_U)
                    cbT = pl.multiple_of(c * (SC_U * LANES), SC_U * LANES)
                    wc = [wv[pl.ds(cb16 + h * LANES, LANES)]
                          for h in range(SC_U // LANES)]
                    for u in range(SC_U):
                        ws = _splat(wc[u // LANES], u % LANES)
                        tb = tbv[pl.ds(cbT + u * LANES, LANES)]
                        a = a + jnp.where(tb <= mid, ws, 0.0)
                    return a

                cnt = lax.fori_loop(
                    0, nch, cnt_body, jnp.zeros((LANES,), jnp.float32))
                pred = cnt >= wb
                return (jnp.where(pred, lo, mid), jnp.where(pred, mid, hi))

            lo, hi = lax.fori_loop(0, SC_ITERS, bis, (lo0, hi0))

            def tail_body(c, ab):
                aw, ad = ab
                cb16 = pl.multiple_of(c * SC_U, SC_U)
                cbT = pl.multiple_of(c * (SC_U * LANES), SC_U * LANES)
                wc = [wv[pl.ds(cb16 + h * LANES, LANES)]
                      for h in range(SC_U // LANES)]
                for u in range(SC_U):
                    ws = _splat(wc[u // LANES], u % LANES)
                    tb = tbv[pl.ds(cbT + u * LANES, LANES)]
                    msk = tb < hi
                    tf = lax.bitcast_convert_type(tb, jnp.float32)
                    aw = aw + jnp.where(msk, ws, 0.0)
                    ad = ad + jnp.where(msk, ws * tf, 0.0)
                return aw, ad

            aw, ad = lax.fori_loop(0, nch, tail_body, (zz, zz))
            t_star = lax.bitcast_convert_type(hi, jnp.float32)
            val = jnp.maximum(ad + t_star * (wb - aw), 0.0)
            v = val / wb
            vb = lax.bitcast_convert_type(v, jnp.int32)
            s = lax.bitcast_convert_type((vb >> 1) + 0x1FBD1DF5, jnp.float32)
            for _ in range(4):
                s = 0.5 * (s + v / s)
            ov[pl.ds(gbase, LANES)] = jnp.where(v > 0, s, 0.0)
            return 0

        lax.fori_loop(0, groups, group_body, 0)
        pltpu.sync_copy(ov, oh.at[wid])

    out = sc_kernel(x0, x1, w, y0f, y1f)
    return out.reshape(B, split)


def kernel(input, weight):
    g = _grid_points()
    y0 = g[:, 0:1]
    y1 = g[:, 1:2]
    parts = []
    if SC_SPLIT < N:
        xt = jnp.swapaxes(input, 1, 2)
        parts.append(_tc_call(y0, y1, xt, weight, N - SC_SPLIT))
    if SC_SPLIT > 0:
        x0 = input[:, :, 0]
        x1 = input[:, :, 1]
        parts.append(_sc_call(x0, x1, weight, g[:, 0], g[:, 1], SC_SPLIT))
    if len(parts) == 1:
        return parts[0]
    return jnp.concatenate(parts, axis=1)


# speedup vs baseline: 1.0858x; 1.0002x over previous
import functools

import jax
import jax.numpy as jnp
from jax import lax
from jax.experimental import pallas as pl
from jax.experimental.pallas import tpu as pltpu
from jax.experimental.pallas import tpu_sc as plsc

M0 = 0.3
SIZE = (40, 40)
LIMS = [[1.0, -1.0], [-1.0, 1.0]]
N = SIZE[0] * SIZE[1]
B = 8

SC_SPLIT = 448
TC_ROWS = 576

NW = 32
LANES = 16
J = N // LANES


def _grid_points():
    e0 = jnp.linspace(LIMS[0][0], LIMS[0][1], SIZE[0])
    e1 = jnp.linspace(LIMS[1][0], LIMS[1][1], SIZE[1])
    g = jnp.stack([jnp.tile(e1, SIZE[0]), jnp.repeat(e0, SIZE[1])], axis=1)
    return g.astype(jnp.float32)


def _tc_body(rows, y0_ref, y1_ref, xt_ref, w_ref, o_ref):
    y0 = y0_ref[:, :]
    y1 = y1_ref[:, :]
    x0 = xt_ref[0, 0:1, :]
    x1 = xt_ref[0, 1:2, :]
    w = w_ref[0, :, :]

    d0 = y0 - x0
    d1 = y1 - x1
    dist = jnp.sqrt(d0 * d0 + d1 * d1)
    t = dist * dist
    t_bits = lax.bitcast_convert_type(t, jnp.int32)

    wb = M0 * jnp.sum(w)

    lo0 = jnp.min(t_bits, axis=1, keepdims=True) - 1
    hi0 = jnp.max(t_bits, axis=1, keepdims=True)

    def body(_, carry):
        lo, hi = carry
        mid = lo + ((hi - lo) >> 1)
        cnt = jnp.sum(jnp.where(t_bits <= mid, w, 0.0), axis=1,
                      keepdims=True)
        pred = cnt >= wb
        return jnp.where(pred, lo, mid), jnp.where(pred, mid, hi)

    lo, hi = lax.fori_loop(0, 20, body, (lo0, hi0))

    t_star = lax.bitcast_convert_type(hi, jnp.float32)
    mask = t_bits < hi
    s_w = jnp.sum(jnp.where(mask, w, 0.0), axis=1, keepdims=True)
    s_d = jnp.sum(jnp.where(mask, w * t, 0.0), axis=1, keepdims=True)
    val = jnp.maximum(s_d + t_star * (wb - s_w), 0.0)
    o_ref[0, :, :] = jnp.sqrt(val / wb)


def _tc_call(y0, y1, xt, weight, n_tc):
    nb = n_tc // TC_ROWS
    out = pl.pallas_call(
        functools.partial(_tc_body, TC_ROWS),
        grid=(B, nb),
        in_specs=[
            pl.BlockSpec((TC_ROWS, 1), lambda b, rb: (rb, 0)),
            pl.BlockSpec((TC_ROWS, 1), lambda b, rb: (rb, 0)),
            pl.BlockSpec((1, 2, N), lambda b, rb: (b, 0, 0)),
            pl.BlockSpec((1, 1, N), lambda b, rb: (b, 0, 0)),
        ],
        out_specs=pl.BlockSpec((1, TC_ROWS, 1), lambda b, rb: (b, rb, 0)),
        out_shape=jax.ShapeDtypeStruct((B, n_tc, 1), jnp.float32),
    )(y0[:n_tc], y1[:n_tc], xt, weight.reshape(B, 1, N))
    return out.reshape(B, n_tc)


def _lane_sum(x):
    iota = lax.broadcasted_iota(jnp.int32, (LANES,), 0)
    dnums = lax.GatherDimensionNumbers(
        offset_dims=(), collapsed_slice_dims=(0,), start_index_map=(0,))
    for k in (8, 4, 2, 1):
        perm = lax.gather(
            x, (iota ^ k)[:, None], dimension_numbers=dnums,
            slice_sizes=(1,), mode=lax.GatherScatterMode.PROMISE_IN_BOUNDS)
        x = x + perm
    return x


def _splat(vec, lane):
    dnums = lax.GatherDimensionNumbers(
        offset_dims=(), collapsed_slice_dims=(0,), start_index_map=(0,))
    idx = jnp.full((LANES, 1), lane, jnp.int32)
    return lax.gather(vec, idx, dimension_numbers=dnums, slice_sizes=(1,),
                      mode=lax.GatherScatterMode.PROMISE_IN_BOUNDS)


SC_U = 32
SC_ITERS = 18


def _sc_call(x0, x1, w, y0f, y1f, split):
    rpw = split // 4
    tc_rows = N - split
    groups = rpw // LANES
    nch = N // SC_U
    mesh = plsc.VectorSubcoreMesh(core_axis_name="c", subcore_axis_name="s")

    @functools.partial(
        pl.kernel, mesh=mesh,
        out_type=jax.ShapeDtypeStruct((NW, rpw), jnp.float32),
        scratch_types=[
            pltpu.VMEM((N,), jnp.float32),
            pltpu.VMEM((N,), jnp.float32),
            pltpu.VMEM((N,), jnp.float32),
            pltpu.VMEM((N * LANES,), jnp.int32),
            pltpu.VMEM((rpw,), jnp.float32),
            pltpu.VMEM((rpw,), jnp.float32),
            pltpu.VMEM((rpw,), jnp.float32),
        ],
    )
    def sc_kernel(x0h, x1h, wh, y0h, y1h, oh, x0v, x1v, wv, tbv, y0v, y1v, ov):
        wid = lax.axis_index("s") * 2 + lax.axis_index("c")
        b = wid // 4
        roff = tc_rows + (wid % 4) * rpw
        pltpu.sync_copy(x0h.at[b], x0v)
        pltpu.sync_copy(x1h.at[b], x1v)
        pltpu.sync_copy(wh.at[b], wv)
        pltpu.sync_copy(y0h.at[pl.ds(roff, rpw)], y0v)
        pltpu.sync_copy(y1h.at[pl.ds(roff, rpw)], y1v)

        acc = jnp.zeros((LANES,), jnp.float32)
        for j in range(J):
            acc = acc + wv[pl.ds(j * LANES, LANES)]
        wb = M0 * _lane_sum(acc)

        def group_body(g, _):
            gbase = pl.multiple_of(g * LANES, LANES)
            y0r = y0v[pl.ds(gbase, LANES)]
            y1r = y1v[pl.ds(gbase, LANES)]

            def dist_body(c, mm):
                tmin, tmax = mm
                cb16 = pl.multiple_of(c * SC_U, SC_U)
                cbT = pl.multiple_of(c * (SC_U * LANES), SC_U * LANES)
                x0c = [x0v[pl.ds(cb16 + h * LANES, LANES)]
                       for h in range(SC_U // LANES)]
                x1c = [x1v[pl.ds(cb16 + h * LANES, LANES)]
                       for h in range(SC_U // LANES)]
                for u in range(SC_U):
                    uu = u % LANES
                    x0s = _splat(x0c[u // LANES], uu)
                    x1s = _splat(x1c[u // LANES], uu)
                    d0 = y0r - x0s
                    d1 = y1r - x1s
                    t = d0 * d0 + d1 * d1
                    tmin = jnp.minimum(tmin, t)
                    tmax = jnp.maximum(tmax, t)
                    tbv[pl.ds(cbT + u * LANES, LANES)] = (
                        lax.bitcast_convert_type(t, jnp.int32))
                return tmin, tmax

            big = jnp.full((LANES,), 3.4e38, jnp.float32)
            zz = jnp.zeros((LANES,), jnp.float32)
            tmin, tmax = lax.fori_loop(0, nch, dist_body, (big, zz))

            lo0 = lax.bitcast_convert_type(tmin, jnp.int32) - 1
            hi0 = lax.bitcast_convert_type(tmax, jnp.int32)

            def bis(_, lh):
                lo, hi = lh
                mid = lo + ((hi - lo) >> 1)

                def cnt_body(c, a):
                    cb16 = pl.multiple_of(c * SC_U, SC_U)
                    cbT = pl.multiple_of(c * (SC_U * LANES), SC_U * LANES)
                    wc = [wv[pl.ds(cb16 + h * LANES, LANES)]
                          for h in range(SC_U // LANES)]
                    for u in range(SC_U):
                        ws = _splat(wc[u // LANES], u % LANES)
                        tb = tbv[pl.ds(cbT + u * LANES, LANES)]
                        a = a + jnp.where(tb <= mid, ws, 0.0)
                    return a

                cnt = lax.fori_loop(
                    0, nch, cnt_body, jnp.zeros((LANES,), jnp.float32))
                pred = cnt >= wb
                return (jnp.where(pred, lo, mid), jnp.where(pred, mid, hi))

            lo, hi = lax.fori_loop(0, SC_ITERS, bis, (lo0, hi0))

            def tail_body(c, ab):
                aw, ad = ab
                cb16 = pl.multiple_of(c * SC_U, SC_U)
                cbT = pl.multiple_of(c * (SC_U * LANES), SC_U * LANES)
                wc = [wv[pl.ds(cb16 + h * LANES, LANES)]
                      for h in range(SC_U // LANES)]
                for u in range(SC_U):
                    ws = _splat(wc[u // LANES], u % LANES)
                    tb = tbv[pl.ds(cbT + u * LANES, LANES)]
                    msk = tb < hi
                    tf = lax.bitcast_convert_type(tb, jnp.float32)
                    aw = aw + jnp.where(msk, ws, 0.0)
                    ad = ad + jnp.where(msk, ws * tf, 0.0)
                return aw, ad

            aw, ad = lax.fori_loop(0, nch, tail_body, (zz, zz))
            t_star = lax.bitcast_convert_type(hi, jnp.float32)
            val = jnp.maximum(ad + t_star * (wb - aw), 0.0)
            v = val / wb
            vb = lax.bitcast_convert_type(v, jnp.int32)
            s = lax.bitcast_convert_type((vb >> 1) + 0x1FBD1DF5, jnp.float32)
            for _ in range(4):
                s = 0.5 * (s + v / s)
            ov[pl.ds(gbase, LANES)] = jnp.where(v > 0, s, 0.0)
            return 0

        lax.fori_loop(0, groups, group_body, 0)
        pltpu.sync_copy(ov, oh.at[wid])

    out = sc_kernel(x0, x1, w, y0f, y1f)
    return out.reshape(B, split)


def kernel(input, weight):
    g = _grid_points()
    y0 = g[:, 0:1]
    y1 = g[:, 1:2]
    parts = []
    if SC_SPLIT < N:
        xt = jnp.swapaxes(input, 1, 2)
        parts.append(_tc_call(y0, y1, xt, weight, N - SC_SPLIT))
    if SC_SPLIT > 0:
        x0 = input[:, :, 0]
        x1 = input[:, :, 1]
        parts.append(_sc_call(x0, x1, weight, g[:, 0], g[:, 1], SC_SPLIT))
    if len(parts) == 1:
        return parts[0]
    return jnp.concatenate(parts, axis=1)


# speedup vs baseline: 1.1859x; 1.0922x over previous
import functools

import jax
import jax.numpy as jnp
from jax import lax
from jax.experimental import pallas as pl
from jax.experimental.pallas import tpu as pltpu
from jax.experimental.pallas import tpu_sc as plsc

M0 = 0.3
SIZE = (40, 40)
LIMS = [[1.0, -1.0], [-1.0, 1.0]]
N = SIZE[0] * SIZE[1]
B = 8

SC_SPLIT = 448
TC_ROWS = 576

NW = 32
LANES = 16
J = N // LANES


def _grid_points():
    e0 = jnp.linspace(LIMS[0][0], LIMS[0][1], SIZE[0])
    e1 = jnp.linspace(LIMS[1][0], LIMS[1][1], SIZE[1])
    g = jnp.stack([jnp.tile(e1, SIZE[0]), jnp.repeat(e0, SIZE[1])], axis=1)
    return g.astype(jnp.float32)


def _tc_body(rows, y0_ref, y1_ref, xt_ref, w_ref, o_ref):
    y0 = y0_ref[:, :]
    y1 = y1_ref[:, :]
    x0 = xt_ref[0, 0:1, :]
    x1 = xt_ref[0, 1:2, :]
    w = w_ref[0, :, :]

    d0 = y0 - x0
    d1 = y1 - x1
    dist = jnp.sqrt(d0 * d0 + d1 * d1)
    t = dist * dist
    t_bits = lax.bitcast_convert_type(t, jnp.int32)

    wb = M0 * jnp.sum(w)

    lo0 = jnp.min(t_bits, axis=1, keepdims=True) - 1
    hi0 = jnp.max(t_bits, axis=1, keepdims=True)

    def body(_, carry):
        lo, hi = carry
        mid = lo + ((hi - lo) >> 1)
        cnt = jnp.sum(jnp.where(t_bits <= mid, w, 0.0), axis=1,
                      keepdims=True)
        pred = cnt >= wb
        return jnp.where(pred, lo, mid), jnp.where(pred, mid, hi)

    lo, hi = lax.fori_loop(0, 20, body, (lo0, hi0))

    t_star = lax.bitcast_convert_type(hi, jnp.float32)
    mask = t_bits < hi
    s_w = jnp.sum(jnp.where(mask, w, 0.0), axis=1, keepdims=True)
    s_d = jnp.sum(jnp.where(mask, w * t, 0.0), axis=1, keepdims=True)
    val = jnp.maximum(s_d + t_star * (wb - s_w), 0.0)
    o_ref[0, :, :] = jnp.sqrt(val / wb)


def _tc_call(y0, y1, xt, weight, n_tc):
    nb = n_tc // TC_ROWS
    out = pl.pallas_call(
        functools.partial(_tc_body, TC_ROWS),
        grid=(B, nb),
        in_specs=[
            pl.BlockSpec((TC_ROWS, 1), lambda b, rb: (rb, 0)),
            pl.BlockSpec((TC_ROWS, 1), lambda b, rb: (rb, 0)),
            pl.BlockSpec((1, 2, N), lambda b, rb: (b, 0, 0)),
            pl.BlockSpec((1, 1, N), lambda b, rb: (b, 0, 0)),
        ],
        out_specs=pl.BlockSpec((1, TC_ROWS, 1), lambda b, rb: (b, rb, 0)),
        out_shape=jax.ShapeDtypeStruct((B, n_tc, 1), jnp.float32),
    )(y0[:n_tc], y1[:n_tc], xt, weight.reshape(B, 1, N))
    return out.reshape(B, n_tc)


def _lane_sum(x):
    iota = lax.broadcasted_iota(jnp.int32, (LANES,), 0)
    dnums = lax.GatherDimensionNumbers(
        offset_dims=(), collapsed_slice_dims=(0,), start_index_map=(0,))
    for k in (8, 4, 2, 1):
        perm = lax.gather(
            x, (iota ^ k)[:, None], dimension_numbers=dnums,
            slice_sizes=(1,), mode=lax.GatherScatterMode.PROMISE_IN_BOUNDS)
        x = x + perm
    return x


def _splat(vec, lane):
    dnums = lax.GatherDimensionNumbers(
        offset_dims=(), collapsed_slice_dims=(0,), start_index_map=(0,))
    idx = jnp.full((LANES, 1), lane, jnp.int32)
    return lax.gather(vec, idx, dimension_numbers=dnums, slice_sizes=(1,),
                      mode=lax.GatherScatterMode.PROMISE_IN_BOUNDS)


SC_U = 32
SC_ITERS = 16


def _sc_call(x0, x1, w, y0f, y1f, split):
    rpw = split // 4
    tc_rows = N - split
    groups = rpw // LANES
    nch = N // SC_U
    mesh = plsc.VectorSubcoreMesh(core_axis_name="c", subcore_axis_name="s")

    @functools.partial(
        pl.kernel, mesh=mesh,
        out_type=jax.ShapeDtypeStruct((NW, rpw), jnp.float32),
        scratch_types=[
            pltpu.VMEM((N,), jnp.float32),
            pltpu.VMEM((N,), jnp.float32),
            pltpu.VMEM((N,), jnp.float32),
            pltpu.VMEM((N * LANES,), jnp.int32),
            pltpu.VMEM((rpw,), jnp.float32),
            pltpu.VMEM((rpw,), jnp.float32),
            pltpu.VMEM((rpw,), jnp.float32),
        ],
    )
    def sc_kernel(x0h, x1h, wh, y0h, y1h, oh, x0v, x1v, wv, tbv, y0v, y1v, ov):
        wid = lax.axis_index("s") * 2 + lax.axis_index("c")
        b = wid // 4
        roff = tc_rows + (wid % 4) * rpw
        pltpu.sync_copy(x0h.at[b], x0v)
        pltpu.sync_copy(x1h.at[b], x1v)
        pltpu.sync_copy(wh.at[b], wv)
        pltpu.sync_copy(y0h.at[pl.ds(roff, rpw)], y0v)
        pltpu.sync_copy(y1h.at[pl.ds(roff, rpw)], y1v)

        acc = jnp.zeros((LANES,), jnp.float32)
        for j in range(J):
            acc = acc + wv[pl.ds(j * LANES, LANES)]
        wb = M0 * _lane_sum(acc)

        def group_body(g, _):
            gbase = pl.multiple_of(g * LANES, LANES)
            y0r = y0v[pl.ds(gbase, LANES)]
            y1r = y1v[pl.ds(gbase, LANES)]

            def dist_body(c, mm):
                tmin, tmax = mm
                cb16 = pl.multiple_of(c * SC_U, SC_U)
                cbT = pl.multiple_of(c * (SC_U * LANES), SC_U * LANES)
                x0c = [x0v[pl.ds(cb16 + h * LANES, LANES)]
                       for h in range(SC_U // LANES)]
                x1c = [x1v[pl.ds(cb16 + h * LANES, LANES)]
                       for h in range(SC_U // LANES)]
                for u in range(SC_U):
                    uu = u % LANES
                    x0s = _splat(x0c[u // LANES], uu)
                    x1s = _splat(x1c[u // LANES], uu)
                    d0 = y0r - x0s
                    d1 = y1r - x1s
                    t = d0 * d0 + d1 * d1
                    tmin = jnp.minimum(tmin, t)
                    tmax = jnp.maximum(tmax, t)
                    tbv[pl.ds(cbT + u * LANES, LANES)] = (
                        lax.bitcast_convert_type(t, jnp.int32))
                return tmin, tmax

            big = jnp.full((LANES,), 3.4e38, jnp.float32)
            zz = jnp.zeros((LANES,), jnp.float32)
            tmin, tmax = lax.fori_loop(0, nch, dist_body, (big, zz))

            lo0 = lax.bitcast_convert_type(tmin, jnp.int32) - 1
            hi0 = lax.bitcast_convert_type(tmax, jnp.int32)

            def bis(_, lh):
                lo, hi = lh
                mid = lo + ((hi - lo) >> 1)

                def cnt_body(c, a):
                    cb16 = pl.multiple_of(c * SC_U, SC_U)
                    cbT = pl.multiple_of(c * (SC_U * LANES), SC_U * LANES)
                    wc = [wv[pl.ds(cb16 + h * LANES, LANES)]
                          for h in range(SC_U // LANES)]
                    for u in range(SC_U):
                        ws = _splat(wc[u // LANES], u % LANES)
                        tb = tbv[pl.ds(cbT + u * LANES, LANES)]
                        a = a + jnp.where(tb <= mid, ws, 0.0)
                    return a

                cnt = lax.fori_loop(
                    0, nch, cnt_body, jnp.zeros((LANES,), jnp.float32))
                pred = cnt >= wb
                return (jnp.where(pred, lo, mid), jnp.where(pred, mid, hi))

            lo, hi = lax.fori_loop(0, SC_ITERS, bis, (lo0, hi0))

            def tail_body(c, ab):
                aw, ad = ab
                cb16 = pl.multiple_of(c * SC_U, SC_U)
                cbT = pl.multiple_of(c * (SC_U * LANES), SC_U * LANES)
                wc = [wv[pl.ds(cb16 + h * LANES, LANES)]
                      for h in range(SC_U // LANES)]
                for u in range(SC_U):
                    ws = _splat(wc[u // LANES], u % LANES)
                    tb = tbv[pl.ds(cbT + u * LANES, LANES)]
                    msk = tb < hi
                    tf = lax.bitcast_convert_type(tb, jnp.float32)
                    aw = aw + jnp.where(msk, ws, 0.0)
                    ad = ad + jnp.where(msk, ws * tf, 0.0)
                return aw, ad

            aw, ad = lax.fori_loop(0, nch, tail_body, (zz, zz))
            t_star = lax.bitcast_convert_type(hi, jnp.float32)
            val = jnp.maximum(ad + t_star * (wb - aw), 0.0)
            v = val / wb
            vb = lax.bitcast_convert_type(v, jnp.int32)
            s = lax.bitcast_convert_type((vb >> 1) + 0x1FBD1DF5, jnp.float32)
            for _ in range(4):
                s = 0.5 * (s + v / s)
            ov[pl.ds(gbase, LANES)] = jnp.where(v > 0, s, 0.0)
            return 0

        lax.fori_loop(0, groups, group_body, 0)
        pltpu.sync_copy(ov, oh.at[wid])

    out = sc_kernel(x0, x1, w, y0f, y1f)
    return out.reshape(B, split)


def kernel(input, weight):
    g = _grid_points()
    y0 = g[:, 0:1]
    y1 = g[:, 1:2]
    parts = []
    if SC_SPLIT < N:
        xt = jnp.swapaxes(input, 1, 2)
        parts.append(_tc_call(y0, y1, xt, weight, N - SC_SPLIT))
    if SC_SPLIT > 0:
        x0 = input[:, :, 0]
        x1 = input[:, :, 1]
        parts.append(_sc_call(x0, x1, weight, g[:, 0], g[:, 1], SC_SPLIT))
    if len(parts) == 1:
        return parts[0]
    return jnp.concatenate(parts, axis=1)


# speedup vs baseline: 1.3155x; 1.1093x over previous
import functools

import jax
import jax.numpy as jnp
from jax import lax
from jax.experimental import pallas as pl
from jax.experimental.pallas import tpu as pltpu
from jax.experimental.pallas import tpu_sc as plsc

M0 = 0.3
SIZE = (40, 40)
LIMS = [[1.0, -1.0], [-1.0, 1.0]]
N = SIZE[0] * SIZE[1]
B = 8

SC_SPLIT = 384
TC_ROWS = 608

NW = 32
LANES = 16
J = N // LANES


def _grid_points():
    e0 = jnp.linspace(LIMS[0][0], LIMS[0][1], SIZE[0])
    e1 = jnp.linspace(LIMS[1][0], LIMS[1][1], SIZE[1])
    g = jnp.stack([jnp.tile(e1, SIZE[0]), jnp.repeat(e0, SIZE[1])], axis=1)
    return g.astype(jnp.float32)


def _tc_body(rows, y0_ref, y1_ref, xt_ref, w_ref, o_ref):
    y0 = y0_ref[:, :]
    y1 = y1_ref[:, :]
    x0 = xt_ref[0, 0:1, :]
    x1 = xt_ref[0, 1:2, :]
    w = w_ref[0, :, :]

    d0 = y0 - x0
    d1 = y1 - x1
    dist = jnp.sqrt(d0 * d0 + d1 * d1)
    t = dist * dist
    t_bits = lax.bitcast_convert_type(t, jnp.int32)

    wb = M0 * jnp.sum(w)

    lo0 = jnp.min(t_bits, axis=1, keepdims=True) - 1
    hi0 = jnp.max(t_bits, axis=1, keepdims=True)

    def body(_, carry):
        lo, hi = carry
        mid = lo + ((hi - lo) >> 1)
        cnt = jnp.sum(jnp.where(t_bits <= mid, w, 0.0), axis=1,
                      keepdims=True)
        pred = cnt >= wb
        return jnp.where(pred, lo, mid), jnp.where(pred, mid, hi)

    lo, hi = lax.fori_loop(0, 16, body, (lo0, hi0))

    t_star = lax.bitcast_convert_type(hi, jnp.float32)
    mask = t_bits < hi
    s_w = jnp.sum(jnp.where(mask, w, 0.0), axis=1, keepdims=True)
    s_d = jnp.sum(jnp.where(mask, w * t, 0.0), axis=1, keepdims=True)
    val = jnp.maximum(s_d + t_star * (wb - s_w), 0.0)
    o_ref[0, :, :] = jnp.sqrt(val / wb)


def _tc_call(y0, y1, xt, weight, n_tc):
    nb = n_tc // TC_ROWS
    out = pl.pallas_call(
        functools.partial(_tc_body, TC_ROWS),
        grid=(B, nb),
        in_specs=[
            pl.BlockSpec((TC_ROWS, 1), lambda b, rb: (rb, 0)),
            pl.BlockSpec((TC_ROWS, 1), lambda b, rb: (rb, 0)),
            pl.BlockSpec((1, 2, N), lambda b, rb: (b, 0, 0)),
            pl.BlockSpec((1, 1, N), lambda b, rb: (b, 0, 0)),
        ],
        out_specs=pl.BlockSpec((1, TC_ROWS, 1), lambda b, rb: (b, rb, 0)),
        out_shape=jax.ShapeDtypeStruct((B, n_tc, 1), jnp.float32),
    )(y0[:n_tc], y1[:n_tc], xt, weight.reshape(B, 1, N))
    return out.reshape(B, n_tc)


def _lane_sum(x):
    iota = lax.broadcasted_iota(jnp.int32, (LANES,), 0)
    dnums = lax.GatherDimensionNumbers(
        offset_dims=(), collapsed_slice_dims=(0,), start_index_map=(0,))
    for k in (8, 4, 2, 1):
        perm = lax.gather(
            x, (iota ^ k)[:, None], dimension_numbers=dnums,
            slice_sizes=(1,), mode=lax.GatherScatterMode.PROMISE_IN_BOUNDS)
        x = x + perm
    return x


def _splat(vec, lane):
    dnums = lax.GatherDimensionNumbers(
        offset_dims=(), collapsed_slice_dims=(0,), start_index_map=(0,))
    idx = jnp.full((LANES, 1), lane, jnp.int32)
    return lax.gather(vec, idx, dimension_numbers=dnums, slice_sizes=(1,),
                      mode=lax.GatherScatterMode.PROMISE_IN_BOUNDS)


SC_U = 32
SC_ITERS = 16


def _sc_call(x0, x1, w, y0f, y1f, split):
    rpw = split // 4
    tc_rows = N - split
    groups = rpw // LANES
    nch = N // SC_U
    mesh = plsc.VectorSubcoreMesh(core_axis_name="c", subcore_axis_name="s")

    @functools.partial(
        pl.kernel, mesh=mesh,
        out_type=jax.ShapeDtypeStruct((NW, rpw), jnp.float32),
        scratch_types=[
            pltpu.VMEM((N,), jnp.float32),
            pltpu.VMEM((N,), jnp.float32),
            pltpu.VMEM((N,), jnp.float32),
            pltpu.VMEM((N * LANES,), jnp.int32),
            pltpu.VMEM((rpw,), jnp.float32),
            pltpu.VMEM((rpw,), jnp.float32),
            pltpu.VMEM((rpw,), jnp.float32),
        ],
    )
    def sc_kernel(x0h, x1h, wh, y0h, y1h, oh, x0v, x1v, wv, tbv, y0v, y1v, ov):
        wid = lax.axis_index("s") * 2 + lax.axis_index("c")
        b = wid // 4
        roff = tc_rows + (wid % 4) * rpw
        pltpu.sync_copy(x0h.at[b], x0v)
        pltpu.sync_copy(x1h.at[b], x1v)
        pltpu.sync_copy(wh.at[b], wv)
        pltpu.sync_copy(y0h.at[pl.ds(roff, rpw)], y0v)
        pltpu.sync_copy(y1h.at[pl.ds(roff, rpw)], y1v)

        acc = jnp.zeros((LANES,), jnp.float32)
        for j in range(J):
            acc = acc + wv[pl.ds(j * LANES, LANES)]
        wb = M0 * _lane_sum(acc)

        def group_body(g, _):
            gbase = pl.multiple_of(g * LANES, LANES)
            y0r = y0v[pl.ds(gbase, LANES)]
            y1r = y1v[pl.ds(gbase, LANES)]

            def dist_body(c, mm):
                tmin, tmax = mm
                cb16 = pl.multiple_of(c * SC_U, SC_U)
                cbT = pl.multiple_of(c * (SC_U * LANES), SC_U * LANES)
                x0c = [x0v[pl.ds(cb16 + h * LANES, LANES)]
                       for h in range(SC_U // LANES)]
                x1c = [x1v[pl.ds(cb16 + h * LANES, LANES)]
                       for h in range(SC_U // LANES)]
                for u in range(SC_U):
                    uu = u % LANES
                    x0s = _splat(x0c[u // LANES], uu)
                    x1s = _splat(x1c[u // LANES], uu)
                    d0 = y0r - x0s
                    d1 = y1r - x1s
                    t = d0 * d0 + d1 * d1
                    tmin = jnp.minimum(tmin, t)
                    tmax = jnp.maximum(tmax, t)
                    tbv[pl.ds(cbT + u * LANES, LANES)] = (
                        lax.bitcast_convert_type(t, jnp.int32))
                return tmin, tmax

            big = jnp.full((LANES,), 3.4e38, jnp.float32)
            zz = jnp.zeros((LANES,), jnp.float32)
            tmin, tmax = lax.fori_loop(0, nch, dist_body, (big, zz))

            lo0 = lax.bitcast_convert_type(tmin, jnp.int32) - 1
            hi0 = lax.bitcast_convert_type(tmax, jnp.int32)

            def bis(_, lh):
                lo, hi = lh
                mid = lo + ((hi - lo) >> 1)

                def cnt_body(c, a):
                    cb16 = pl.multiple_of(c * SC_U, SC_U)
                    cbT = pl.multiple_of(c * (SC_U * LANES), SC_U * LANES)
                    wc = [wv[pl.ds(cb16 + h * LANES, LANES)]
                          for h in range(SC_U // LANES)]
                    for u in range(SC_U):
                        ws = _splat(wc[u // LANES], u % LANES)
                        tb = tbv[pl.ds(cbT + u * LANES, LANES)]
                        a = a + jnp.where(tb <= mid, ws, 0.0)
                    return a

                cnt = lax.fori_loop(
                    0, nch, cnt_body, jnp.zeros((LANES,), jnp.float32))
                pred = cnt >= wb
                return (jnp.where(pred, lo, mid), jnp.where(pred, mid, hi))

            lo, hi = lax.fori_loop(0, SC_ITERS, bis, (lo0, hi0))

            def tail_body(c, ab):
                aw, ad = ab
                cb16 = pl.multiple_of(c * SC_U, SC_U)
                cbT = pl.multiple_of(c * (SC_U * LANES), SC_U * LANES)
                wc = [wv[pl.ds(cb16 + h * LANES, LANES)]
                      for h in range(SC_U // LANES)]
                for u in range(SC_U):
                    ws = _splat(wc[u // LANES], u % LANES)
                    tb = tbv[pl.ds(cbT + u * LANES, LANES)]
                    msk = tb < hi
                    tf = lax.bitcast_convert_type(tb, jnp.float32)
                    aw = aw + jnp.where(msk, ws, 0.0)
                    ad = ad + jnp.where(msk, ws * tf, 0.0)
                return aw, ad

            aw, ad = lax.fori_loop(0, nch, tail_body, (zz, zz))
            t_star = lax.bitcast_convert_type(hi, jnp.float32)
            val = jnp.maximum(ad + t_star * (wb - aw), 0.0)
            v = val / wb
            vb = lax.bitcast_convert_type(v, jnp.int32)
            s = lax.bitcast_convert_type((vb >> 1) + 0x1FBD1DF5, jnp.float32)
            for _ in range(4):
                s = 0.5 * (s + v / s)
            ov[pl.ds(gbase, LANES)] = jnp.where(v > 0, s, 0.0)
            return 0

        lax.fori_loop(0, groups, group_body, 0)
        pltpu.sync_copy(ov, oh.at[wid])

    out = sc_kernel(x0, x1, w, y0f, y1f)
    return out.reshape(B, split)


def kernel(input, weight):
    g = _grid_points()
    y0 = g[:, 0:1]
    y1 = g[:, 1:2]
    parts = []
    if SC_SPLIT < N:
        xt = jnp.swapaxes(input, 1, 2)
        parts.append(_tc_call(y0, y1, xt, weight, N - SC_SPLIT))
    if SC_SPLIT > 0:
        x0 = input[:, :, 0]
        x1 = input[:, :, 1]
        parts.append(_sc_call(x0, x1, weight, g[:, 0], g[:, 1], SC_SPLIT))
    if len(parts) == 1:
        return parts[0]
    return jnp.concatenate(parts, axis=1)


# speedup vs baseline: 1.4305x; 1.0874x over previous
import functools

import jax
import jax.numpy as jnp
from jax import lax
from jax.experimental import pallas as pl
from jax.experimental.pallas import tpu as pltpu
from jax.experimental.pallas import tpu_sc as plsc

M0 = 0.3
SIZE = (40, 40)
LIMS = [[1.0, -1.0], [-1.0, 1.0]]
N = SIZE[0] * SIZE[1]
B = 8

SC_SPLIT = 384
TC_ROWS = 608

NW = 32
LANES = 16
J = N // LANES


def _grid_points():
    e0 = jnp.linspace(LIMS[0][0], LIMS[0][1], SIZE[0])
    e1 = jnp.linspace(LIMS[1][0], LIMS[1][1], SIZE[1])
    g = jnp.stack([jnp.tile(e1, SIZE[0]), jnp.repeat(e0, SIZE[1])], axis=1)
    return g.astype(jnp.float32)


def _tc_body(rows, y0_ref, y1_ref, xt_ref, w_ref, o_ref):
    y0 = y0_ref[:, :]
    y1 = y1_ref[:, :]
    x0 = xt_ref[0, 0:1, :]
    x1 = xt_ref[0, 1:2, :]
    w = w_ref[0, :, :]

    d0 = y0 - x0
    d1 = y1 - x1
    dist = jnp.sqrt(d0 * d0 + d1 * d1)
    t = dist * dist
    t_bits = lax.bitcast_convert_type(t, jnp.int32)

    wb = M0 * jnp.sum(w)

    lo0 = jnp.min(t_bits, axis=1, keepdims=True) - 1
    hi0 = jnp.max(t_bits, axis=1, keepdims=True)

    def body(_, carry):
        lo, hi = carry
        mid = lo + ((hi - lo) >> 1)
        cnt = jnp.sum(jnp.where(t_bits <= mid, w, 0.0), axis=1,
                      keepdims=True)
        pred = cnt >= wb
        return jnp.where(pred, lo, mid), jnp.where(pred, mid, hi)

    lo, hi = lax.fori_loop(0, 14, body, (lo0, hi0))

    t_star = lax.bitcast_convert_type(hi, jnp.float32)
    mask = t_bits < hi
    s_w = jnp.sum(jnp.where(mask, w, 0.0), axis=1, keepdims=True)
    s_d = jnp.sum(jnp.where(mask, w * t, 0.0), axis=1, keepdims=True)
    val = jnp.maximum(s_d + t_star * (wb - s_w), 0.0)
    o_ref[0, :, :] = jnp.sqrt(val / wb)


def _tc_call(y0, y1, xt, weight, n_tc):
    nb = n_tc // TC_ROWS
    out = pl.pallas_call(
        functools.partial(_tc_body, TC_ROWS),
        grid=(B, nb),
        in_specs=[
            pl.BlockSpec((TC_ROWS, 1), lambda b, rb: (rb, 0)),
            pl.BlockSpec((TC_ROWS, 1), lambda b, rb: (rb, 0)),
            pl.BlockSpec((1, 2, N), lambda b, rb: (b, 0, 0)),
            pl.BlockSpec((1, 1, N), lambda b, rb: (b, 0, 0)),
        ],
        out_specs=pl.BlockSpec((1, TC_ROWS, 1), lambda b, rb: (b, rb, 0)),
        out_shape=jax.ShapeDtypeStruct((B, n_tc, 1), jnp.float32),
    )(y0[:n_tc], y1[:n_tc], xt, weight.reshape(B, 1, N))
    return out.reshape(B, n_tc)


def _lane_sum(x):
    iota = lax.broadcasted_iota(jnp.int32, (LANES,), 0)
    dnums = lax.GatherDimensionNumbers(
        offset_dims=(), collapsed_slice_dims=(0,), start_index_map=(0,))
    for k in (8, 4, 2, 1):
        perm = lax.gather(
            x, (iota ^ k)[:, None], dimension_numbers=dnums,
            slice_sizes=(1,), mode=lax.GatherScatterMode.PROMISE_IN_BOUNDS)
        x = x + perm
    return x


def _splat(vec, lane):
    dnums = lax.GatherDimensionNumbers(
        offset_dims=(), collapsed_slice_dims=(0,), start_index_map=(0,))
    idx = jnp.full((LANES, 1), lane, jnp.int32)
    return lax.gather(vec, idx, dimension_numbers=dnums, slice_sizes=(1,),
                      mode=lax.GatherScatterMode.PROMISE_IN_BOUNDS)


SC_U = 32
SC_ITERS = 14


def _sc_call(x0, x1, w, y0f, y1f, split):
    rpw = split // 4
    tc_rows = N - split
    groups = rpw // LANES
    nch = N // SC_U
    mesh = plsc.VectorSubcoreMesh(core_axis_name="c", subcore_axis_name="s")

    @functools.partial(
        pl.kernel, mesh=mesh,
        out_type=jax.ShapeDtypeStruct((NW, rpw), jnp.float32),
        scratch_types=[
            pltpu.VMEM((N,), jnp.float32),
            pltpu.VMEM((N,), jnp.float32),
            pltpu.VMEM((N,), jnp.float32),
            pltpu.VMEM((N * LANES,), jnp.int32),
            pltpu.VMEM((rpw,), jnp.float32),
            pltpu.VMEM((rpw,), jnp.float32),
            pltpu.VMEM((rpw,), jnp.float32),
        ],
    )
    def sc_kernel(x0h, x1h, wh, y0h, y1h, oh, x0v, x1v, wv, tbv, y0v, y1v, ov):
        wid = lax.axis_index("s") * 2 + lax.axis_index("c")
        b = wid // 4
        roff = tc_rows + (wid % 4) * rpw
        pltpu.sync_copy(x0h.at[b], x0v)
        pltpu.sync_copy(x1h.at[b], x1v)
        pltpu.sync_copy(wh.at[b], wv)
        pltpu.sync_copy(y0h.at[pl.ds(roff, rpw)], y0v)
        pltpu.sync_copy(y1h.at[pl.ds(roff, rpw)], y1v)

        acc = jnp.zeros((LANES,), jnp.float32)
        for j in range(J):
            acc = acc + wv[pl.ds(j * LANES, LANES)]
        wb = M0 * _lane_sum(acc)

        def group_body(g, _):
            gbase = pl.multiple_of(g * LANES, LANES)
            y0r = y0v[pl.ds(gbase, LANES)]
            y1r = y1v[pl.ds(gbase, LANES)]

            def dist_body(c, mm):
                tmin, tmax = mm
                cb16 = pl.multiple_of(c * SC_U, SC_U)
                cbT = pl.multiple_of(c * (SC_U * LANES), SC_U * LANES)
                x0c = [x0v[pl.ds(cb16 + h * LANES, LANES)]
                       for h in range(SC_U // LANES)]
                x1c = [x1v[pl.ds(cb16 + h * LANES, LANES)]
                       for h in range(SC_U // LANES)]
                for u in range(SC_U):
                    uu = u % LANES
                    x0s = _splat(x0c[u // LANES], uu)
                    x1s = _splat(x1c[u // LANES], uu)
                    d0 = y0r - x0s
                    d1 = y1r - x1s
                    t = d0 * d0 + d1 * d1
                    tmin = jnp.minimum(tmin, t)
                    tmax = jnp.maximum(tmax, t)
                    tbv[pl.ds(cbT + u * LANES, LANES)] = (
                        lax.bitcast_convert_type(t, jnp.int32))
                return tmin, tmax

            big = jnp.full((LANES,), 3.4e38, jnp.float32)
            zz = jnp.zeros((LANES,), jnp.float32)
            tmin, tmax = lax.fori_loop(0, nch, dist_body, (big, zz))

            lo0 = lax.bitcast_convert_type(tmin, jnp.int32) - 1
            hi0 = lax.bitcast_convert_type(tmax, jnp.int32)

            def bis(_, lh):
                lo, hi = lh
                mid = lo + ((hi - lo) >> 1)

                def cnt_body(c, a):
                    cb16 = pl.multiple_of(c * SC_U, SC_U)
                    cbT = pl.multiple_of(c * (SC_U * LANES), SC_U * LANES)
                    wc = [wv[pl.ds(cb16 + h * LANES, LANES)]
                          for h in range(SC_U // LANES)]
                    for u in range(SC_U):
                        ws = _splat(wc[u // LANES], u % LANES)
                        tb = tbv[pl.ds(cbT + u * LANES, LANES)]
                        a = a + jnp.where(tb <= mid, ws, 0.0)
                    return a

                cnt = lax.fori_loop(
                    0, nch, cnt_body, jnp.zeros((LANES,), jnp.float32))
                pred = cnt >= wb
                return (jnp.where(pred, lo, mid), jnp.where(pred, mid, hi))

            lo, hi = lax.fori_loop(0, SC_ITERS, bis, (lo0, hi0))

            def tail_body(c, ab):
                aw, ad = ab
                cb16 = pl.multiple_of(c * SC_U, SC_U)
                cbT = pl.multiple_of(c * (SC_U * LANES), SC_U * LANES)
                wc = [wv[pl.ds(cb16 + h * LANES, LANES)]
                      for h in range(SC_U // LANES)]
                for u in range(SC_U):
                    ws = _splat(wc[u // LANES], u % LANES)
                    tb = tbv[pl.ds(cbT + u * LANES, LANES)]
                    msk = tb < hi
                    tf = lax.bitcast_convert_type(tb, jnp.float32)
                    aw = aw + jnp.where(msk, ws, 0.0)
                    ad = ad + jnp.where(msk, ws * tf, 0.0)
                return aw, ad

            aw, ad = lax.fori_loop(0, nch, tail_body, (zz, zz))
            t_star = lax.bitcast_convert_type(hi, jnp.float32)
            val = jnp.maximum(ad + t_star * (wb - aw), 0.0)
            v = val / wb
            vb = lax.bitcast_convert_type(v, jnp.int32)
            s = lax.bitcast_convert_type((vb >> 1) + 0x1FBD1DF5, jnp.float32)
            for _ in range(4):
                s = 0.5 * (s + v / s)
            ov[pl.ds(gbase, LANES)] = jnp.where(v > 0, s, 0.0)
            return 0

        lax.fori_loop(0, groups, group_body, 0)
        pltpu.sync_copy(ov, oh.at[wid])

    out = sc_kernel(x0, x1, w, y0f, y1f)
    return out.reshape(B, split)


def kernel(input, weight):
    g = _grid_points()
    y0 = g[:, 0:1]
    y1 = g[:, 1:2]
    parts = []
    if SC_SPLIT < N:
        xt = jnp.swapaxes(input, 1, 2)
        parts.append(_tc_call(y0, y1, xt, weight, N - SC_SPLIT))
    if SC_SPLIT > 0:
        x0 = input[:, :, 0]
        x1 = input[:, :, 1]
        parts.append(_sc_call(x0, x1, weight, g[:, 0], g[:, 1], SC_SPLIT))
    if len(parts) == 1:
        return parts[0]
    return jnp.concatenate(parts, axis=1)
